# baseline (speedup 1.0000x reference)
import functools

import jax
import jax.numpy as jnp
from jax import lax
from jax.experimental import pallas as pl
from jax.experimental.pallas import tpu as pltpu
from jax.experimental.pallas import tpu_sc as plsc

N = 10000
F = 128
FH = F // 2
E = 320000
NC = 2
NS = 16
NW = NC * NS
CH = 128
CPW = 80
CPT = 160
CPH = 4
CPP = CPT // CPH
EP = CPT * CH * NS
ACC_N = 10240
RPT = ACC_N // NS
NBUF = 2
BR = 10000
GRID = N // BR

assert CPW * CH * NW == EP


@functools.cache
def _mesh():
    return plsc.VectorSubcoreMesh(
        core_axis_name="c", subcore_axis_name="s", num_cores=NC,
        num_subcores=NS,
    )


def _sc_degrees_body(src_hbm, dst_hbm, out_hbm, src_idx, dst_idx, ones_v,
                     sacc, dacc):
    c = lax.axis_index("c")
    s = lax.axis_index("s")
    wid = s * NC + c
    pltpu.sync_copy(src_hbm.at[pl.ds(wid * CPW, CPW)], src_idx)
    pltpu.sync_copy(dst_hbm.at[pl.ds(wid * CPW, CPW)], dst_idx)
    for i in range(CH):
        ones_v[i, :] = jnp.zeros((16,), jnp.float32)
    for k in range(RPT // CH):
        pltpu.sync_copy(ones_v, sacc.at[pl.ds(s * RPT + k * CH, CH)])
        pltpu.sync_copy(ones_v, dacc.at[pl.ds(s * RPT + k * CH, CH)])
    for i in range(CH):
        ones_v[i, :] = jnp.ones((16,), jnp.float32)
    plsc.subcore_barrier()

    @pl.loop(0, CPW)
    def _(j):
        pltpu.sync_copy(ones_v, sacc.at[src_idx.at[j]], add=True)
        pltpu.sync_copy(ones_v, dacc.at[dst_idx.at[j]], add=True)

    plsc.subcore_barrier()
    pltpu.sync_copy(sacc.at[pl.ds(s * RPT, RPT)],
                    out_hbm.at[c, 0, pl.ds(s * RPT, RPT)])
    pltpu.sync_copy(dacc.at[pl.ds(s * RPT, RPT)],
                    out_hbm.at[c, 1, pl.ds(s * RPT, RPT)])


@functools.cache
def _sc_degrees_kernel():
    return pl.kernel(
        _sc_degrees_body,
        out_type=jax.ShapeDtypeStruct((NC, 2, ACC_N, 16), jnp.float32),
        mesh=_mesh(),
        scratch_types=[
            pltpu.VMEM((CPW, CH), jnp.int32),
            pltpu.VMEM((CPW, CH), jnp.int32),
            pltpu.VMEM((CH, 16), jnp.float32),
            pltpu.VMEM_SHARED((ACC_N, 16), jnp.float32),
            pltpu.VMEM_SHARED((ACC_N, 16), jnp.float32),
        ],
        compiler_params=pltpu.CompilerParams(use_tc_tiling_on_sc=False),
    )


def _sc_degrees(src_d, dst_d):
    return _sc_degrees_kernel()(src_d, dst_d)


def _sc_agg_body(h_hbm, src_hbm, dst_hbm, out_hbm, src_idx, dst_idx, bufs,
                 acc, hsp, *sems):
    NSL = 2 * NBUF
    gsems = sems[:NSL]
    ssems = sems[NSL:]
    c = lax.axis_index("c")
    s = lax.axis_index("s")
    for i in range(CH):
        for jj in range(FH // 16):
            bufs[0, i, pl.ds(jj * 16, 16)] = jnp.zeros((16,), jnp.float32)
    for k in range(RPT // CH):
        pltpu.sync_copy(bufs.at[0], acc.at[pl.ds(s * RPT + k * CH, CH)])
    for k in range(RPT // CH):
        pltpu.sync_copy(h_hbm.at[c].at[pl.ds(s * RPT + k * CH, CH)],
                        hsp.at[pl.ds(s * RPT + k * CH, CH)])
    plsc.subcore_barrier()

    def gather(j, t):
        pltpu.async_copy(hsp.at[src_idx.at[j]], bufs.at[t], gsems[t])

    def gather_wait(j, t):
        pltpu.make_async_copy(
            hsp.at[src_idx.at[j]], bufs.at[t], gsems[t]
        ).wait()

    def scatter(j, t):
        pltpu.async_copy(bufs.at[t], acc.at[dst_idx.at[j]], ssems[t],
                         add=True)

    def scatter_wait(j, t):
        pltpu.make_async_copy(
            bufs.at[t], acc.at[dst_idx.at[j]], ssems[t]
        ).wait()

    for h in range(CPH):
        pltpu.sync_copy(src_hbm.at[pl.ds(s * CPT + h * CPP, CPP)], src_idx)
        pltpu.sync_copy(dst_hbm.at[pl.ds(s * CPT + h * CPP, CPP)], dst_idx)

        for t in range(NBUF):
            gather(t, t)
        for t in range(NBUF):
            gather_wait(t, t)
            scatter(t, t)
            gather(t + NBUF, t + NBUF)
        for t in range(NBUF, NSL):
            gather_wait(t, t)
            scatter(t, t)
            scatter_wait(t - NBUF, t - NBUF)
            gather(t + NBUF, t - NBUF)

        @pl.loop(NSL, CPP, step=NSL)
        def _(jj):
            for t in range(NSL):
                j = jj + t
                t2 = (t + NBUF) % NSL
                gather_wait(j, t)
                scatter(j, t)
                scatter_wait(j - NBUF, t2)

                @pl.when(j + NBUF < CPP)
                def _():
                    gather(j + NBUF, t2)

        for t in range(NBUF, NSL):
            scatter_wait(CPP - NSL + t, t)

    plsc.subcore_barrier()
    pltpu.sync_copy(acc.at[pl.ds(s * RPT, RPT)],
                    out_hbm.at[c, pl.ds(s * RPT, RPT)])


@functools.cache
def _sc_agg_kernel():
    return pl.kernel(
        _sc_agg_body,
        out_type=jax.ShapeDtypeStruct((NC, ACC_N, FH), jnp.float32),
        mesh=_mesh(),
        scratch_types=[
            pltpu.VMEM((CPP, CH), jnp.int32),
            pltpu.VMEM((CPP, CH), jnp.int32),
            pltpu.VMEM((2 * NBUF, CH, FH), jnp.float32),
            pltpu.VMEM_SHARED((ACC_N, FH), jnp.float32),
            pltpu.VMEM_SHARED((ACC_N, FH), jnp.float32),
        ] + [pltpu.SemaphoreType.DMA] * (4 * NBUF),
        compiler_params=pltpu.CompilerParams(use_tc_tiling_on_sc=False),
    )


def _sc_agg(h_split, src_g, dst_d):
    return _sc_agg_kernel()(h_split, src_g, dst_d)


def _split_store(out_ref, val):
    out_ref[0] = val[:, :FH]
    out_ref[1] = val[:, FH:]


def _deg_norms(deg_ref):
    d = deg_ref[...]
    od = d[0, 0] + d[1, 0]
    idg = d[0, 1] + d[1, 1]
    on = lax.rsqrt(jnp.maximum(od[:, :1], 1.0))
    inn = lax.rsqrt(jnp.maximum(idg[:, :1], 1.0))
    return on, inn


def _norms_body(deg_ref, x_ref, xs_ref):
    on, _ = _deg_norms(deg_ref)
    _split_store(xs_ref, x_ref[...] * on)


def _norms(degp, x):
    return pl.pallas_call(
        _norms_body,
        grid=(GRID,),
        in_specs=[
            pl.BlockSpec((NC, 2, BR, 16), lambda i: (0, 0, i, 0)),
            pl.BlockSpec((BR, F), lambda i: (i, 0)),
        ],
        out_specs=pl.BlockSpec((NC, BR, FH), lambda i: (0, i, 0)),
        out_shape=jax.ShapeDtypeStruct((NC, ACC_N, FH), jnp.float32),
    )(degp, x)


def _layer12_body(p_ref, deg_ref, w_ref, b_ref, hs_ref):
    on, inn = _deg_norms(deg_ref)
    p = jnp.concatenate([p_ref[0], p_ref[1]], axis=-1)
    t = p * inn
    h = jnp.dot(t, w_ref[...], preferred_element_type=jnp.float32) + b_ref[...]
    _split_store(hs_ref, jnp.maximum(h, 0.0) * on)


def _layer12(p, degp, W, b):
    return pl.pallas_call(
        _layer12_body,
        grid=(GRID,),
        in_specs=[
            pl.BlockSpec((NC, BR, FH), lambda i: (0, i, 0)),
            pl.BlockSpec((NC, 2, BR, 16), lambda i: (0, 0, i, 0)),
            pl.BlockSpec((F, F), lambda i: (0, 0)),
            pl.BlockSpec((1, F), lambda i: (0, 0)),
        ],
        out_specs=pl.BlockSpec((NC, BR, FH), lambda i: (0, i, 0)),
        out_shape=jax.ShapeDtypeStruct((NC, ACC_N, FH), jnp.float32),
    )(p, degp, W, b.reshape(1, F))


def _layer3_body(p_ref, deg_ref, w_ref, b_ref, g_ref, be_ref, h_ref, hg_ref,
                 acc_ref):
    i = pl.program_id(0)
    _, inn = _deg_norms(deg_ref)
    p = jnp.concatenate([p_ref[0], p_ref[1]], axis=-1)
    t = p * inn
    h = jnp.dot(t, w_ref[...], preferred_element_type=jnp.float32) + b_ref[...]
    h_ref[...] = h
    part = jnp.sum(h, axis=0, keepdims=True)

    @pl.when(i == 0)
    def _():
        acc_ref[...] = part

    @pl.when(i > 0)
    def _():
        acc_ref[...] += part

    @pl.when(i == GRID - 1)
    def _():
        hm = acc_ref[...] / N
        mu = jnp.mean(hm, axis=-1, keepdims=True)
        var = jnp.mean((hm - mu) ** 2, axis=-1, keepdims=True)
        hg_ref[...] = (hm - mu) * lax.rsqrt(var + 1e-5) * g_ref[...] + be_ref[...]


def _layer3(p, degp, W, b, gamma, beta):
    return pl.pallas_call(
        _layer3_body,
        grid=(GRID,),
        in_specs=[
            pl.BlockSpec((NC, BR, FH), lambda i: (0, i, 0)),
            pl.BlockSpec((NC, 2, BR, 16), lambda i: (0, 0, i, 0)),
            pl.BlockSpec((F, F), lambda i: (0, 0)),
            pl.BlockSpec((1, F), lambda i: (0, 0)),
            pl.BlockSpec((1, F), lambda i: (0, 0)),
            pl.BlockSpec((1, F), lambda i: (0, 0)),
        ],
        out_specs=[
            pl.BlockSpec((BR, F), lambda i: (i, 0)),
            pl.BlockSpec((1, F), lambda i: (0, 0)),
        ],
        out_shape=[
            jax.ShapeDtypeStruct((N, F), jnp.float32),
            jax.ShapeDtypeStruct((1, F), jnp.float32),
        ],
        scratch_shapes=[pltpu.VMEM((1, F), jnp.float32)],
        compiler_params=pltpu.CompilerParams(
            dimension_semantics=("arbitrary",)
        ),
    )(p, degp, W, b.reshape(1, F), gamma.reshape(1, F), beta.reshape(1, F))


def kernel(x, edge_index, W1, b1, W2, b2, W3, b3, gamma, beta):
    src = edge_index[0]
    dst = edge_index[1]
    pad = EP - E
    pad_dummy = (jnp.arange(pad, dtype=jnp.int32) % (ACC_N - N)) + N
    src_g = jnp.concatenate([src, jnp.zeros((pad,), jnp.int32)]).reshape(-1, CH)
    src_d = jnp.concatenate([src, pad_dummy]).reshape(-1, CH)
    dst_d = jnp.concatenate([dst, pad_dummy]).reshape(-1, CH)

    degp = _sc_degrees(src_d, dst_d)
    xs = _norms(degp, x)
    p1 = _sc_agg(xs, src_g, dst_d)
    h1 = _layer12(p1, degp, W1, b1)
    p2 = _sc_agg(h1, src_g, dst_d)
    h2 = _layer12(p2, degp, W2, b2)
    p3 = _sc_agg(h2, src_g, dst_d)
    h3, hg = _layer3(p3, degp, W3, b3, gamma, beta)
    return (hg, h3)


# speedup vs baseline: 1.0184x; 1.0184x over previous
import functools

import jax
import jax.numpy as jnp
from jax import lax
from jax.experimental import pallas as pl
from jax.experimental.pallas import tpu as pltpu
from jax.experimental.pallas import tpu_sc as plsc

N = 10000
F = 128
FH = F // 2
E = 320000
NC = 2
NS = 16
NW = NC * NS
CH = 128
CPW = 80
CPT = 160
CPH = 4
CPP = CPT // CPH
EP = CPT * CH * NS
ACC_N = 10240
RPT = ACC_N // NS
NBUF = 2
BR = 5000
GRID = N // BR

assert CPW * CH * NW == EP


@functools.cache
def _mesh():
    return plsc.VectorSubcoreMesh(
        core_axis_name="c", subcore_axis_name="s", num_cores=NC,
        num_subcores=NS,
    )


def _sc_degrees_body(src_hbm, dst_hbm, out_hbm, src_idx, dst_idx, ones_v,
                     sacc, dacc):
    c = lax.axis_index("c")
    s = lax.axis_index("s")
    wid = s * NC + c
    pltpu.sync_copy(src_hbm.at[pl.ds(wid * CPW, CPW)], src_idx)
    pltpu.sync_copy(dst_hbm.at[pl.ds(wid * CPW, CPW)], dst_idx)
    for i in range(CH):
        ones_v[i, :] = jnp.zeros((16,), jnp.float32)
    for k in range(RPT // CH):
        pltpu.sync_copy(ones_v, sacc.at[pl.ds(s * RPT + k * CH, CH)])
        pltpu.sync_copy(ones_v, dacc.at[pl.ds(s * RPT + k * CH, CH)])
    for i in range(CH):
        ones_v[i, :] = jnp.ones((16,), jnp.float32)
    plsc.subcore_barrier()

    @pl.loop(0, CPW)
    def _(j):
        pltpu.sync_copy(ones_v, sacc.at[src_idx.at[j]], add=True)
        pltpu.sync_copy(ones_v, dacc.at[dst_idx.at[j]], add=True)

    plsc.subcore_barrier()
    pltpu.sync_copy(sacc.at[pl.ds(s * RPT, RPT)],
                    out_hbm.at[c, 0, pl.ds(s * RPT, RPT)])
    pltpu.sync_copy(dacc.at[pl.ds(s * RPT, RPT)],
                    out_hbm.at[c, 1, pl.ds(s * RPT, RPT)])


@functools.cache
def _sc_degrees_kernel():
    return pl.kernel(
        _sc_degrees_body,
        out_type=jax.ShapeDtypeStruct((NC, 2, ACC_N, 16), jnp.float32),
        mesh=_mesh(),
        scratch_types=[
            pltpu.VMEM((CPW, CH), jnp.int32),
            pltpu.VMEM((CPW, CH), jnp.int32),
            pltpu.VMEM((CH, 16), jnp.float32),
            pltpu.VMEM_SHARED((ACC_N, 16), jnp.float32),
            pltpu.VMEM_SHARED((ACC_N, 16), jnp.float32),
        ],
        compiler_params=pltpu.CompilerParams(use_tc_tiling_on_sc=False),
    )


def _sc_degrees(src_d, dst_d):
    return _sc_degrees_kernel()(src_d, dst_d)


def _sc_agg_body(h_hbm, src_hbm, dst_hbm, out_hbm, src_idx, dst_idx, bufs,
                 acc, hsp, *sems):
    NSL = 2 * NBUF
    gsems = sems[:NSL]
    ssems = sems[NSL:]
    c = lax.axis_index("c")
    s = lax.axis_index("s")
    for i in range(CH):
        for jj in range(FH // 16):
            bufs[0, i, pl.ds(jj * 16, 16)] = jnp.zeros((16,), jnp.float32)
    for k in range(RPT // CH):
        pltpu.sync_copy(bufs.at[0], acc.at[pl.ds(s * RPT + k * CH, CH)])
    for k in range(RPT // CH):
        pltpu.sync_copy(h_hbm.at[c].at[pl.ds(s * RPT + k * CH, CH)],
                        hsp.at[pl.ds(s * RPT + k * CH, CH)])
    plsc.subcore_barrier()

    def gather(j, t):
        pltpu.async_copy(hsp.at[src_idx.at[j]], bufs.at[t], gsems[t])

    def gather_wait(j, t):
        pltpu.make_async_copy(
            hsp.at[src_idx.at[j]], bufs.at[t], gsems[t]
        ).wait()

    def scatter(j, t):
        pltpu.async_copy(bufs.at[t], acc.at[dst_idx.at[j]], ssems[t],
                         add=True)

    def scatter_wait(j, t):
        pltpu.make_async_copy(
            bufs.at[t], acc.at[dst_idx.at[j]], ssems[t]
        ).wait()

    for h in range(CPH):
        pltpu.sync_copy(src_hbm.at[pl.ds(s * CPT + h * CPP, CPP)], src_idx)
        pltpu.sync_copy(dst_hbm.at[pl.ds(s * CPT + h * CPP, CPP)], dst_idx)

        for t in range(NBUF):
            gather(t, t)
        for t in range(NBUF):
            gather_wait(t, t)
            scatter(t, t)
            gather(t + NBUF, t + NBUF)
        for t in range(NBUF, NSL):
            gather_wait(t, t)
            scatter(t, t)
            scatter_wait(t - NBUF, t - NBUF)
            gather(t + NBUF, t - NBUF)

        @pl.loop(NSL, CPP, step=NSL)
        def _(jj):
            for t in range(NSL):
                j = jj + t
                t2 = (t + NBUF) % NSL
                gather_wait(j, t)
                scatter(j, t)
                scatter_wait(j - NBUF, t2)

                @pl.when(j + NBUF < CPP)
                def _():
                    gather(j + NBUF, t2)

        for t in range(NBUF, NSL):
            scatter_wait(CPP - NSL + t, t)

    plsc.subcore_barrier()
    pltpu.sync_copy(acc.at[pl.ds(s * RPT, RPT)],
                    out_hbm.at[c, pl.ds(s * RPT, RPT)])


@functools.cache
def _sc_agg_kernel():
    return pl.kernel(
        _sc_agg_body,
        out_type=jax.ShapeDtypeStruct((NC, ACC_N, FH), jnp.float32),
        mesh=_mesh(),
        scratch_types=[
            pltpu.VMEM((CPP, CH), jnp.int32),
            pltpu.VMEM((CPP, CH), jnp.int32),
            pltpu.VMEM((2 * NBUF, CH, FH), jnp.float32),
            pltpu.VMEM_SHARED((ACC_N, FH), jnp.float32),
            pltpu.VMEM_SHARED((ACC_N, FH), jnp.float32),
        ] + [pltpu.SemaphoreType.DMA] * (4 * NBUF),
        compiler_params=pltpu.CompilerParams(use_tc_tiling_on_sc=False),
    )


def _sc_agg(h_split, src_g, dst_d):
    return _sc_agg_kernel()(h_split, src_g, dst_d)


def _split_store(out_ref, val):
    out_ref[0] = val[:, :FH]
    out_ref[1] = val[:, FH:]


def _deg_norms(deg_ref):
    d = deg_ref[...]
    od = d[0, 0] + d[1, 0]
    idg = d[0, 1] + d[1, 1]
    on = lax.rsqrt(jnp.maximum(od[:, :1], 1.0))
    inn = lax.rsqrt(jnp.maximum(idg[:, :1], 1.0))
    return on, inn


def _norms_body(deg_ref, x_ref, xs_ref):
    on, _ = _deg_norms(deg_ref)
    _split_store(xs_ref, x_ref[...] * on)


def _norms(degp, x):
    return pl.pallas_call(
        _norms_body,
        grid=(GRID,),
        in_specs=[
            pl.BlockSpec((NC, 2, BR, 16), lambda i: (0, 0, i, 0)),
            pl.BlockSpec((BR, F), lambda i: (i, 0)),
        ],
        out_specs=pl.BlockSpec((NC, BR, FH), lambda i: (0, i, 0)),
        out_shape=jax.ShapeDtypeStruct((NC, ACC_N, FH), jnp.float32),
    )(degp, x)


def _layer12_body(p_ref, deg_ref, w_ref, b_ref, hs_ref):
    on, inn = _deg_norms(deg_ref)
    p = jnp.concatenate([p_ref[0], p_ref[1]], axis=-1)
    t = p * inn
    h = jnp.dot(t, w_ref[...], preferred_element_type=jnp.float32) + b_ref[...]
    _split_store(hs_ref, jnp.maximum(h, 0.0) * on)


def _layer12(p, degp, W, b):
    return pl.pallas_call(
        _layer12_body,
        grid=(GRID,),
        in_specs=[
            pl.BlockSpec((NC, BR, FH), lambda i: (0, i, 0)),
            pl.BlockSpec((NC, 2, BR, 16), lambda i: (0, 0, i, 0)),
            pl.BlockSpec((F, F), lambda i: (0, 0)),
            pl.BlockSpec((1, F), lambda i: (0, 0)),
        ],
        out_specs=pl.BlockSpec((NC, BR, FH), lambda i: (0, i, 0)),
        out_shape=jax.ShapeDtypeStruct((NC, ACC_N, FH), jnp.float32),
    )(p, degp, W, b.reshape(1, F))


def _layer3_body(p_ref, deg_ref, w_ref, b_ref, g_ref, be_ref, h_ref, hg_ref,
                 acc_ref):
    i = pl.program_id(0)
    _, inn = _deg_norms(deg_ref)
    p = jnp.concatenate([p_ref[0], p_ref[1]], axis=-1)
    t = p * inn
    h = jnp.dot(t, w_ref[...], preferred_element_type=jnp.float32) + b_ref[...]
    h_ref[...] = h
    part = jnp.sum(h, axis=0, keepdims=True)

    @pl.when(i == 0)
    def _():
        acc_ref[...] = part

    @pl.when(i > 0)
    def _():
        acc_ref[...] += part

    @pl.when(i == GRID - 1)
    def _():
        hm = acc_ref[...] / N
        mu = jnp.mean(hm, axis=-1, keepdims=True)
        var = jnp.mean((hm - mu) ** 2, axis=-1, keepdims=True)
        hg_ref[...] = (hm - mu) * lax.rsqrt(var + 1e-5) * g_ref[...] + be_ref[...]


def _layer3(p, degp, W, b, gamma, beta):
    return pl.pallas_call(
        _layer3_body,
        grid=(GRID,),
        in_specs=[
            pl.BlockSpec((NC, BR, FH), lambda i: (0, i, 0)),
            pl.BlockSpec((NC, 2, BR, 16), lambda i: (0, 0, i, 0)),
            pl.BlockSpec((F, F), lambda i: (0, 0)),
            pl.BlockSpec((1, F), lambda i: (0, 0)),
            pl.BlockSpec((1, F), lambda i: (0, 0)),
            pl.BlockSpec((1, F), lambda i: (0, 0)),
        ],
        out_specs=[
            pl.BlockSpec((BR, F), lambda i: (i, 0)),
            pl.BlockSpec((1, F), lambda i: (0, 0)),
        ],
        out_shape=[
            jax.ShapeDtypeStruct((N, F), jnp.float32),
            jax.ShapeDtypeStruct((1, F), jnp.float32),
        ],
        scratch_shapes=[pltpu.VMEM((1, F), jnp.float32)],
        compiler_params=pltpu.CompilerParams(
            dimension_semantics=("arbitrary",)
        ),
    )(p, degp, W, b.reshape(1, F), gamma.reshape(1, F), beta.reshape(1, F))


def kernel(x, edge_index, W1, b1, W2, b2, W3, b3, gamma, beta):
    src = edge_index[0]
    dst = edge_index[1]
    pad = EP - E
    pad_dummy = (jnp.arange(pad, dtype=jnp.int32) % (ACC_N - N)) + N
    src_g = jnp.concatenate([src, jnp.zeros((pad,), jnp.int32)]).reshape(-1, CH)
    src_d = jnp.concatenate([src, pad_dummy]).reshape(-1, CH)
    dst_d = jnp.concatenate([dst, pad_dummy]).reshape(-1, CH)

    degp = _sc_degrees(src_d, dst_d)
    xs = _norms(degp, x)
    p1 = _sc_agg(xs, src_g, dst_d)
    h1 = _layer12(p1, degp, W1, b1)
    p2 = _sc_agg(h1, src_g, dst_d)
    h2 = _layer12(p2, degp, W2, b2)
    p3 = _sc_agg(h2, src_g, dst_d)
    h3, hg = _layer3(p3, degp, W3, b3, gamma, beta)
    return (hg, h3)
